# baseline (speedup 1.0000x reference)
import functools

import jax
import jax.numpy as jnp
from jax import lax
from jax.experimental import pallas as pl
from jax.experimental.pallas import tpu as pltpu
from jax.experimental.pallas import tpu_sc as plsc

N_B = 16
N_MAX = 625
N_NODES = 10000
N_EDGES = 160000
N_PRED = 100000
HID = 128
ATT = 16
NDF = 16

NW = 32
NODE_PAD = 10240
T_ROWS = (N_B + 1) * N_MAX
ZROW = N_NODES + 304
COL = 64
QOFF = 10240
EDGE_CH = 128
EDGE_NCH = 80
EDGE_PAD = 16 * EDGE_NCH * EDGE_CH
H_ROWS = 10240
H_TILE = H_ROWS // 16
PRED_PER_W = 3200
PRED_PAD = NW * PRED_PER_W

_MESH = dict(core_axis_name="c", subcore_axis_name="s")


def _nf_body(a_ref, w_ref, b_ref, o_ref):
    i = pl.program_id(0)

    @pl.when(i < N_B)
    def _():
        o_ref[0] = (
            jnp.dot(a_ref[0, :, 0, :], w_ref[...],
                    preferred_element_type=jnp.float32)
            + b_ref[...]
        )

    @pl.when(i >= N_B)
    def _():
        o_ref[0] = jnp.zeros((N_MAX, 2 * HID), jnp.float32)


def _node_feats(A_pad, Wc, bc):
    return pl.pallas_call(
        _nf_body,
        grid=(N_B + 1,),
        in_specs=[
            pl.BlockSpec((1, N_MAX, 1, N_MAX),
                         lambda i: (jnp.minimum(i, N_B - 1), 0, 0, 0)),
            pl.BlockSpec((N_MAX, 2 * HID), lambda i: (0, 0)),
            pl.BlockSpec((1, 2 * HID), lambda i: (0, 0)),
        ],
        out_specs=pl.BlockSpec((1, N_MAX, 2 * HID), lambda i: (i, 0, 0)),
        out_shape=jax.ShapeDtypeStruct((N_B + 1, N_MAX, 2 * HID),
                                       jnp.float32),
    )(A_pad, Wc, bc)


def _gather_rows(table, idx3):
    per_w = NODE_PAD // NW
    ch = 40
    n_ch = per_w // ch
    nb = 4
    lag = 2

    @functools.partial(
        pl.kernel,
        out_type=jax.ShapeDtypeStruct((NODE_PAD, 2 * HID), jnp.float32),
        mesh=plsc.VectorSubcoreMesh(**_MESH),
        scratch_types=[
            pltpu.VMEM((n_ch, ch), jnp.int32),
            pltpu.VMEM((ch, 2 * HID), jnp.float32),
            pltpu.VMEM((ch, 2 * HID), jnp.float32),
            pltpu.VMEM((ch, 2 * HID), jnp.float32),
            pltpu.VMEM((ch, 2 * HID), jnp.float32),
            pltpu.SemaphoreType.DMA,
            pltpu.SemaphoreType.DMA,
            pltpu.SemaphoreType.DMA,
            pltpu.SemaphoreType.DMA,
            pltpu.SemaphoreType.DMA,
            pltpu.SemaphoreType.DMA,
            pltpu.SemaphoreType.DMA,
            pltpu.SemaphoreType.DMA,
        ],
    )
    def k(table_hbm, idx_hbm, out_hbm, idx_v, r0, r1, r2, r3,
          g0, g1, g2, g3, o0, o1, o2, o3):
        wid = lax.axis_index("s") * 2 + lax.axis_index("c")
        rows = [r0, r1, r2, r3]
        gsem = [g0, g1, g2, g3]
        osem = [o0, o1, o2, o3]
        pltpu.sync_copy(idx_hbm.at[wid], idx_v)
        for b in range(nb - lag):
            pltpu.async_copy(table_hbm.at[idx_v.at[b]], rows[b], gsem[b])

        def body(ci, carry):
            for b in range(nb):
                c = ci * nb + b
                pltpu.make_async_copy(table_hbm.at[idx_v.at[0]], rows[b],
                                      gsem[b]).wait()
                base = pl.multiple_of(wid * per_w + c * ch, 8)
                pltpu.async_copy(rows[b], out_hbm.at[pl.ds(base, ch)],
                                 osem[b])

                bf = (b + nb - lag) % nb
                nxt = c + nb - lag

                @pl.when(jnp.logical_and(c >= lag, nxt < n_ch))
                def _():
                    pltpu.make_async_copy(
                        rows[bf], out_hbm.at[pl.ds(0, ch)], osem[bf]).wait()
                    pltpu.async_copy(table_hbm.at[idx_v.at[nxt]], rows[bf],
                                     gsem[bf])

                @pl.when(jnp.logical_and(c < lag, nxt < n_ch))
                def _():
                    pltpu.async_copy(table_hbm.at[idx_v.at[nxt]], rows[bf],
                                     gsem[bf])
            return carry

        lax.fori_loop(0, n_ch // nb, body, 0)
        for b in range(nb):
            pltpu.make_async_copy(rows[b], out_hbm.at[pl.ds(0, ch)],
                                  osem[b]).wait()

    return k(table, idx3)


def _qr_body(s_ref, ai_ref, ni_ref, w_ref, b_ref, q0_ref, q1_ref):
    s = s_ref[...][:, :HID]
    rows = s.shape[0]
    oha = (lax.broadcasted_iota(jnp.int32, (rows, ATT), 1) == ai_ref[...])
    ohn = (lax.broadcasted_iota(jnp.int32, (rows, NDF), 1) == ni_ref[...])
    x = jnp.concatenate(
        [s, oha.astype(jnp.float32), ohn.astype(jnp.float32)], axis=1)
    qr = (jnp.dot(x, w_ref[...], preferred_element_type=jnp.float32)
          + b_ref[...])
    q0_ref[...] = qr[:, :2 * COL]
    q1_ref[...] = qr[:, 2 * COL:]


def _qr(state_cat, att2, nd2, Wqr, bqr):
    R = 1024
    K = HID + ATT + NDF
    qspec = pl.BlockSpec((R, 2 * COL), lambda i: (i, 0))
    qshape = jax.ShapeDtypeStruct((QOFF, 2 * COL), jnp.float32)
    return pl.pallas_call(
        _qr_body,
        grid=(QOFF // R,),
        in_specs=[
            pl.BlockSpec((R, 2 * HID), lambda i: (i, 0)),
            pl.BlockSpec((R, 1), lambda i: (i, 0)),
            pl.BlockSpec((R, 1), lambda i: (i, 0)),
            pl.BlockSpec((K, 2 * HID), lambda i: (0, 0)),
            pl.BlockSpec((1, 2 * HID), lambda i: (0, 0)),
        ],
        out_specs=[qspec, qspec],
        out_shape=[qshape, qshape],
    )(state_cat, att2, nd2, Wqr, bqr)


def _edge_accumulate(qv, rv, sadj, dadj, draw):
    nb = 2
    n_g = EDGE_NCH // nb

    @functools.partial(
        pl.kernel,
        out_type=jax.ShapeDtypeStruct((H_ROWS, HID), jnp.float32),
        mesh=plsc.VectorSubcoreMesh(**_MESH),
        compiler_params=pltpu.CompilerParams(use_tc_tiling_on_sc=False),
        scratch_types=[
            pltpu.VMEM((EDGE_NCH, EDGE_CH), jnp.int32),
            pltpu.VMEM((EDGE_NCH, EDGE_CH), jnp.int32),
            pltpu.VMEM((EDGE_NCH, EDGE_CH), jnp.int32),
            pltpu.VMEM((EDGE_CH, COL), jnp.float32),
            pltpu.VMEM((EDGE_CH, COL), jnp.float32),
            pltpu.VMEM((EDGE_CH, COL), jnp.float32),
            pltpu.VMEM((EDGE_CH, COL), jnp.float32),
            pltpu.VMEM((EDGE_CH, COL), jnp.float32),
            pltpu.VMEM((EDGE_CH, COL), jnp.float32),
            pltpu.VMEM_SHARED((H_ROWS, COL), jnp.float32),
            pltpu.SemaphoreType.DMA,
            pltpu.SemaphoreType.DMA,
            pltpu.SemaphoreType.DMA,
            pltpu.SemaphoreType.DMA,
            pltpu.SemaphoreType.DMA,
            pltpu.SemaphoreType.DMA,
        ],
    )
    def k(q_hbm, r_hbm, sadj_hbm, dadj_hbm, draw_hbm, out_hbm,
          sidx, didx, dids,
          qb0, qb1, rb0, rb1, ob0, ob1, hacc,
          sq0, sq1, sr0, sr1, ss0, ss1):
        qb = [qb0, qb1]
        rb = [rb0, rb1]
        ob = [ob0, ob1]
        sq = [sq0, sq1]
        sr = [sr0, sr1]
        ss = [ss0, ss1]
        cid = lax.axis_index("c")
        sid = lax.axis_index("s")

        pltpu.sync_copy(sadj_hbm.at[cid * 16 + sid], sidx)
        pltpu.sync_copy(dadj_hbm.at[cid * 16 + sid], didx)
        pltpu.sync_copy(draw_hbm.at[sid], dids)

        zero = jnp.zeros((16,), jnp.float32)

        def zrow(r, carry):
            for g in range(COL // 16):
                qb0[r, pl.ds(g * 16, 16)] = zero
            return carry

        lax.fori_loop(0, EDGE_CH, zrow, 0)

        def zcp(i, carry):
            pltpu.sync_copy(qb0, hacc.at[pl.ds(sid * H_TILE + i * EDGE_CH,
                                               EDGE_CH)])
            return carry

        lax.fori_loop(0, H_TILE // EDGE_CH, zcp, 0)
        plsc.subcore_barrier()

        def gathers(ci, b):
            pltpu.async_copy(q_hbm.at[sidx.at[ci]], qb[b], sq[b])
            pltpu.async_copy(r_hbm.at[didx.at[ci]], rb[b], sr[b])

        for b in range(nb):
            gathers(b, b)

        def body(g, carry):
            for b in range(nb):
                ci = g * nb + b
                pltpu.make_async_copy(q_hbm.at[sidx.at[0]], qb[b],
                                      sq[b]).wait()
                pltpu.make_async_copy(r_hbm.at[didx.at[0]], rb[b],
                                      sr[b]).wait()

                @pl.when(g > 0)
                def _():
                    pltpu.make_async_copy(ob[b], hacc.at[dids.at[0]],
                                          ss[b]).wait()

                obb, qbb, rbb = ob[b], qb[b], rb[b]

                @plsc.parallel_loop(0, EDGE_CH, 1, unroll=8)
                def _(r):
                    for gg in range(COL // 16):
                        sl = pl.ds(gg * 16, 16)
                        obb[r, sl] = jnp.maximum(qbb[r, sl] + rbb[r, sl], 0.0)

                @pl.when(g < n_g - 1)
                def _():
                    gathers(ci + nb, b)

                pltpu.async_copy(ob[b], hacc.at[dids.at[ci]], ss[b], add=True)
            return carry

        lax.fori_loop(0, n_g, body, 0)
        for b in range(nb):
            pltpu.make_async_copy(ob[b], hacc.at[dids.at[0]], ss[b]).wait()
        plsc.subcore_barrier()

        def ocp(i, carry):
            off = sid * H_TILE + i * EDGE_CH
            pltpu.sync_copy(hacc.at[pl.ds(off, EDGE_CH)],
                            out_hbm.at[pl.ds(off, EDGE_CH),
                                       pl.ds(cid * COL, COL)])
            return carry

        lax.fori_loop(0, H_TILE // EDGE_CH, ocp, 0)

    return k(qv, rv, sadj, dadj, draw)


def _lstm_body(h_ref, st_ref, wm2_ref, wih_ref, whh_ref, bg_ref,
               wo_ref, bo_ref, o_ref):
    sm = jnp.dot(h_ref[...], wm2_ref[...], preferred_element_type=jnp.float32)
    st = st_ref[...]
    gates = (jnp.dot(sm, wih_ref[...], preferred_element_type=jnp.float32)
             + jnp.dot(st[:, :HID], whh_ref[...],
                       preferred_element_type=jnp.float32)
             + bg_ref[...])
    i = jax.nn.sigmoid(gates[:, 0 * HID:1 * HID])
    f = jax.nn.sigmoid(gates[:, 1 * HID:2 * HID])
    g = jnp.tanh(gates[:, 2 * HID:3 * HID])
    o = jax.nn.sigmoid(gates[:, 3 * HID:4 * HID])
    c = f * st[:, HID:] + i * g
    sn = o * jnp.tanh(c)
    o_ref[...] = (jnp.dot(sn, wo_ref[...], preferred_element_type=jnp.float32)
                  + bo_ref[...])


def _lstm_head(hout, state_cat, Wm2, Wih, Whh, bg, Wo2, bo2):
    R = 1000
    return pl.pallas_call(
        _lstm_body,
        grid=(N_NODES // R,),
        in_specs=[
            pl.BlockSpec((R, HID), lambda i: (i, 0)),
            pl.BlockSpec((R, 2 * HID), lambda i: (i, 0)),
            pl.BlockSpec((HID, HID), lambda i: (0, 0)),
            pl.BlockSpec((HID, 4 * HID), lambda i: (0, 0)),
            pl.BlockSpec((HID, 4 * HID), lambda i: (0, 0)),
            pl.BlockSpec((1, 4 * HID), lambda i: (0, 0)),
            pl.BlockSpec((HID, 8), lambda i: (0, 0)),
            pl.BlockSpec((1, 8), lambda i: (0, 0)),
        ],
        out_specs=pl.BlockSpec((R, 8), lambda i: (i, 0)),
        out_shape=jax.ShapeDtypeStruct((N_NODES, 8), jnp.float32),
    )(hout, state_cat, Wm2, Wih, Whh, bg, Wo2, bo2)


def _pred_head(s8f, i0, i1):
    groups = PRED_PER_W // 16

    @functools.partial(
        pl.kernel,
        out_type=jax.ShapeDtypeStruct((PRED_PAD,), jnp.float32),
        mesh=plsc.VectorSubcoreMesh(**_MESH),
        compiler_params=pltpu.CompilerParams(needs_layout_passes=False),
        scratch_types=[
            pltpu.VMEM((N_NODES * 8,), jnp.float32),
            pltpu.VMEM((PRED_PER_W,), jnp.int32),
            pltpu.VMEM((PRED_PER_W,), jnp.int32),
            pltpu.VMEM((PRED_PER_W,), jnp.float32),
        ],
    )
    def k(s8_hbm, i0_hbm, i1_hbm, out_hbm, s8_v, i0_v, i1_v, ob):
        wid = lax.axis_index("s") * 2 + lax.axis_index("c")
        base = pl.multiple_of(wid * PRED_PER_W, 8)
        pltpu.sync_copy(s8_hbm, s8_v)
        pltpu.sync_copy(i0_hbm.at[pl.ds(base, PRED_PER_W)], i0_v)
        pltpu.sync_copy(i1_hbm.at[pl.ds(base, PRED_PER_W)], i1_v)

        def body(j, carry):
            sl = pl.ds(j * 16, 16)
            a = plsc.load_gather(s8_v, [i0_v[sl] * 8])
            b = plsc.load_gather(s8_v, [i1_v[sl] * 8 + 1])
            ob[sl] = a - b
            return carry

        lax.fori_loop(0, groups, body, 0)
        pltpu.sync_copy(ob, out_hbm.at[pl.ds(base, PRED_PER_W)])

    return k(s8f, i0, i1)


def kernel(A_pad, edges, node_idx_gnn, node_idx_feat, att_idx, node_dist,
           W_in, b_in, W_inc, b_inc, Wm1, bm1, Wm2, bm2,
           W_ih, W_hh, b_ih, b_hh, Wo, bo):
    f32 = jnp.float32
    nd = node_dist.reshape(-1).astype(jnp.int32)
    att = att_idx.astype(jnp.int32)

    Wc = jnp.concatenate([W_in, W_inc], axis=1)
    bc = jnp.concatenate([b_in, b_inc]).reshape(1, 2 * HID)
    A4 = A_pad.reshape(N_B, N_MAX, 1, N_MAX)
    table = _node_feats(A4, Wc, bc).reshape(T_ROWS, 2 * HID)

    nif = node_idx_feat.astype(jnp.int32)
    idx0 = jnp.where(nif == 0, ZROW, nif - 1)
    idx3 = jnp.concatenate(
        [idx0, jnp.full((NODE_PAD - N_NODES,), ZROW, jnp.int32)]
    ).reshape(NW, 8, 40)
    state_cat = _gather_rows(table, idx3)

    W0 = Wm1[:HID]
    Wqr = jnp.concatenate([
        jnp.concatenate([W0, Wm1[HID:HID + ATT],
                         Wm1[HID + 2 * ATT:HID + 2 * ATT + NDF]], axis=0),
        jnp.concatenate([-W0, Wm1[HID + ATT:HID + 2 * ATT],
                         Wm1[HID + 2 * ATT + NDF:]], axis=0),
    ], axis=1)
    bqr = jnp.concatenate([bm1, jnp.zeros((HID,), f32)]).reshape(1, 2 * HID)
    ipad = jnp.zeros((NODE_PAD - N_NODES,), jnp.int32)
    att_p = jnp.concatenate([att, ipad]).reshape(-1, 1)
    nd_p = jnp.concatenate([nd, ipad]).reshape(-1, 1)
    qp, rp = _qr(state_cat, att_p, nd_p, Wqr, bqr)
    qv = qp.reshape(2 * QOFF, COL)
    rv = rp.reshape(2 * QOFF, COL)

    pad_e = jnp.full((1, EDGE_PAD - N_EDGES), N_NODES, jnp.int32)
    edT = edges.T.astype(jnp.int32)
    src3 = jnp.concatenate([edT[0:1], pad_e], axis=1) \
              .reshape(16, EDGE_NCH, EDGE_CH)
    dst3 = jnp.concatenate([edT[1:2], pad_e], axis=1) \
              .reshape(16, EDGE_NCH, EDGE_CH)
    sadj = jnp.concatenate([2 * src3, 2 * src3 + 1], axis=0)
    dadj = jnp.concatenate([2 * dst3, 2 * dst3 + 1], axis=0)
    hout = _edge_accumulate(qv, rv, sadj, dadj, dst3)

    bg = (b_ih + b_hh).reshape(1, 4 * HID)
    Wo2 = jnp.concatenate([Wo, Wo] + [jnp.zeros((HID, 1), f32)] * 6, axis=1)
    bo2 = jnp.concatenate([bo, jnp.zeros((7,), f32)]).reshape(1, 8)
    s8 = _lstm_head(hout, state_cat, Wm2, W_ih, W_hh, bg, Wo2, bo2)

    pad_p = jnp.zeros((1, PRED_PAD - N_PRED), jnp.int32)
    giT = node_idx_gnn.T.astype(jnp.int32)
    i0 = jnp.concatenate([giT[0:1], pad_p], axis=1).reshape(-1)
    i1 = jnp.concatenate([giT[1:2], pad_p], axis=1).reshape(-1)
    out = _pred_head(s8.reshape(-1), i0, i1)
    return out[:N_PRED].reshape(-1, 1)


# speedup vs baseline: 1.0603x; 1.0603x over previous
import functools

import jax
import jax.numpy as jnp
from jax import lax
from jax.experimental import pallas as pl
from jax.experimental.pallas import tpu as pltpu
from jax.experimental.pallas import tpu_sc as plsc

N_B = 16
N_MAX = 625
N_NODES = 10000
N_EDGES = 160000
N_PRED = 100000
HID = 128
ATT = 16
NDF = 16

NW = 32
NODE_PAD = 10240
T_ROWS = (N_B + 1) * N_MAX
ZROW = N_NODES + 304
COL = 64
QOFF = 11000
EDGE_CH = 128
EDGE_NCH = 80
EDGE_PAD = 16 * EDGE_NCH * EDGE_CH
H_ROWS = 10240
H_TILE = H_ROWS // 16
PRED_PER_W = 3200
PRED_PAD = NW * PRED_PER_W

_MESH = dict(core_axis_name="c", subcore_axis_name="s")


def _nf_body(a_ref, w_ref, b_ref, o_ref):
    i = pl.program_id(0)

    @pl.when(i < N_B)
    def _():
        o_ref[0] = (
            jnp.dot(a_ref[0, :, 0, :], w_ref[...],
                    preferred_element_type=jnp.float32)
            + b_ref[...]
        )

    @pl.when(i >= N_B)
    def _():
        o_ref[0] = jnp.zeros((N_MAX, 2 * HID), jnp.float32)


def _node_feats(A_pad, Wc, bc):
    return pl.pallas_call(
        _nf_body,
        grid=(N_B + 1,),
        in_specs=[
            pl.BlockSpec((1, N_MAX, 1, N_MAX),
                         lambda i: (jnp.minimum(i, N_B - 1), 0, 0, 0)),
            pl.BlockSpec((N_MAX, 2 * HID), lambda i: (0, 0)),
            pl.BlockSpec((1, 2 * HID), lambda i: (0, 0)),
        ],
        out_specs=pl.BlockSpec((1, N_MAX, 2 * HID), lambda i: (i, 0, 0)),
        out_shape=jax.ShapeDtypeStruct((N_B + 1, N_MAX, 2 * HID),
                                       jnp.float32),
    )(A_pad, Wc, bc)


def _gather_rows(table, idx3):
    per_w = NODE_PAD // NW
    ch = 40
    n_ch = per_w // ch
    nb = 4
    lag = 2

    @functools.partial(
        pl.kernel,
        out_type=jax.ShapeDtypeStruct((NODE_PAD, 2 * HID), jnp.float32),
        mesh=plsc.VectorSubcoreMesh(**_MESH),
        scratch_types=[
            pltpu.VMEM((n_ch, ch), jnp.int32),
            pltpu.VMEM((ch, 2 * HID), jnp.float32),
            pltpu.VMEM((ch, 2 * HID), jnp.float32),
            pltpu.VMEM((ch, 2 * HID), jnp.float32),
            pltpu.VMEM((ch, 2 * HID), jnp.float32),
            pltpu.SemaphoreType.DMA,
            pltpu.SemaphoreType.DMA,
            pltpu.SemaphoreType.DMA,
            pltpu.SemaphoreType.DMA,
            pltpu.SemaphoreType.DMA,
            pltpu.SemaphoreType.DMA,
            pltpu.SemaphoreType.DMA,
            pltpu.SemaphoreType.DMA,
        ],
    )
    def k(table_hbm, idx_hbm, out_hbm, idx_v, r0, r1, r2, r3,
          g0, g1, g2, g3, o0, o1, o2, o3):
        wid = lax.axis_index("s") * 2 + lax.axis_index("c")
        rows = [r0, r1, r2, r3]
        gsem = [g0, g1, g2, g3]
        osem = [o0, o1, o2, o3]
        pltpu.sync_copy(idx_hbm.at[wid], idx_v)
        for b in range(nb - lag):
            pltpu.async_copy(table_hbm.at[idx_v.at[b]], rows[b], gsem[b])

        def body(ci, carry):
            for b in range(nb):
                c = ci * nb + b
                pltpu.make_async_copy(table_hbm.at[idx_v.at[0]], rows[b],
                                      gsem[b]).wait()
                base = pl.multiple_of(wid * per_w + c * ch, 8)
                pltpu.async_copy(rows[b], out_hbm.at[pl.ds(base, ch)],
                                 osem[b])

                bf = (b + nb - lag) % nb
                nxt = c + nb - lag

                @pl.when(jnp.logical_and(c >= lag, nxt < n_ch))
                def _():
                    pltpu.make_async_copy(
                        rows[bf], out_hbm.at[pl.ds(0, ch)], osem[bf]).wait()
                    pltpu.async_copy(table_hbm.at[idx_v.at[nxt]], rows[bf],
                                     gsem[bf])

                @pl.when(jnp.logical_and(c < lag, nxt < n_ch))
                def _():
                    pltpu.async_copy(table_hbm.at[idx_v.at[nxt]], rows[bf],
                                     gsem[bf])
            return carry

        lax.fori_loop(0, n_ch // nb, body, 0)
        for b in range(nb):
            pltpu.make_async_copy(rows[b], out_hbm.at[pl.ds(0, ch)],
                                  osem[b]).wait()

    return k(table, idx3)


def _qr_body(s_ref, ai_ref, ni_ref, w_ref, b_ref,
             q0_ref, q1_ref, r0_ref, r1_ref):
    i = pl.program_id(0)

    @pl.when(i < 10)
    def _():
        s = s_ref[...][:, :HID]
        rows = s.shape[0]
        oha = (lax.broadcasted_iota(jnp.int32, (rows, ATT), 1) == ai_ref[...])
        ohn = (lax.broadcasted_iota(jnp.int32, (rows, NDF), 1) == ni_ref[...])
        x = jnp.concatenate(
            [s, oha.astype(jnp.float32), ohn.astype(jnp.float32)], axis=1)
        qr = (jnp.dot(x, w_ref[...], preferred_element_type=jnp.float32)
              + b_ref[...])
        q0_ref[...] = qr[:, 0 * COL:1 * COL]
        q1_ref[...] = qr[:, 1 * COL:2 * COL]
        r0_ref[...] = qr[:, 2 * COL:3 * COL]
        r1_ref[...] = qr[:, 3 * COL:4 * COL]

    @pl.when(i >= 10)
    def _():
        q0_ref[...] = jnp.zeros_like(q0_ref)
        q1_ref[...] = jnp.zeros_like(q1_ref)
        r0_ref[...] = jnp.zeros_like(r0_ref)
        r1_ref[...] = jnp.zeros_like(r1_ref)


def _qr(state_cat, att2, nd2, Wqr, bqr):
    R = 1000
    K = HID + ATT + NDF
    qspec = pl.BlockSpec((R, COL), lambda i: (i, 0))
    qshape = jax.ShapeDtypeStruct((QOFF, COL), jnp.float32)
    return pl.pallas_call(
        _qr_body,
        grid=(QOFF // R,),
        in_specs=[
            pl.BlockSpec((R, 2 * HID), lambda i: (jnp.minimum(i, 9), 0)),
            pl.BlockSpec((R, 1), lambda i: (jnp.minimum(i, 9), 0)),
            pl.BlockSpec((R, 1), lambda i: (jnp.minimum(i, 9), 0)),
            pl.BlockSpec((K, 2 * HID), lambda i: (0, 0)),
            pl.BlockSpec((1, 2 * HID), lambda i: (0, 0)),
        ],
        out_specs=[qspec, qspec, qspec, qspec],
        out_shape=[qshape, qshape, qshape, qshape],
    )(state_cat, att2, nd2, Wqr, bqr)


def _edge_accumulate(q0, q1, r0, r1, src3, dst3):
    nb = 4
    lag = 2
    n_g = EDGE_NCH // nb

    @functools.partial(
        pl.kernel,
        out_type=jax.ShapeDtypeStruct((H_ROWS, HID), jnp.float32),
        mesh=plsc.VectorSubcoreMesh(**_MESH),
        compiler_params=pltpu.CompilerParams(use_tc_tiling_on_sc=False),
        scratch_types=[
            pltpu.VMEM((EDGE_NCH, EDGE_CH), jnp.int32),
            pltpu.VMEM((EDGE_NCH, EDGE_CH), jnp.int32),
            pltpu.VMEM((EDGE_CH, COL), jnp.float32),
            pltpu.VMEM((EDGE_CH, COL), jnp.float32),
            pltpu.VMEM((EDGE_CH, COL), jnp.float32),
            pltpu.VMEM((EDGE_CH, COL), jnp.float32),
            pltpu.VMEM((EDGE_CH, COL), jnp.float32),
            pltpu.VMEM((EDGE_CH, COL), jnp.float32),
            pltpu.VMEM((EDGE_CH, COL), jnp.float32),
            pltpu.VMEM((EDGE_CH, COL), jnp.float32),
            pltpu.VMEM_SHARED((H_ROWS, COL), jnp.float32),
            pltpu.SemaphoreType.DMA,
            pltpu.SemaphoreType.DMA,
            pltpu.SemaphoreType.DMA,
            pltpu.SemaphoreType.DMA,
            pltpu.SemaphoreType.DMA,
            pltpu.SemaphoreType.DMA,
            pltpu.SemaphoreType.DMA,
            pltpu.SemaphoreType.DMA,
            pltpu.SemaphoreType.DMA,
            pltpu.SemaphoreType.DMA,
            pltpu.SemaphoreType.DMA,
            pltpu.SemaphoreType.DMA,
        ],
    )
    def k(q0_hbm, q1_hbm, r0_hbm, r1_hbm, src_hbm, dst_hbm, out_hbm,
          sidx, didx,
          qb0, qb1, qb2, qb3, rb0, rb1, rb2, rb3, hacc,
          sq0, sq1, sq2, sq3, sr0, sr1, sr2, sr3, ss0, ss1, ss2, ss3):
        qb = [qb0, qb1, qb2, qb3]
        rb = [rb0, rb1, rb2, rb3]
        sq = [sq0, sq1, sq2, sq3]
        sr = [sr0, sr1, sr2, sr3]
        ss = [ss0, ss1, ss2, ss3]
        cid = lax.axis_index("c")
        sid = lax.axis_index("s")

        pltpu.sync_copy(src_hbm.at[sid], sidx)
        pltpu.sync_copy(dst_hbm.at[sid], didx)

        zero = jnp.zeros((16,), jnp.float32)

        def zrow(r, carry):
            for g in range(COL // 16):
                qb0[r, pl.ds(g * 16, 16)] = zero
            return carry

        lax.fori_loop(0, EDGE_CH, zrow, 0)

        def zcp(i, carry):
            pltpu.sync_copy(qb0, hacc.at[pl.ds(sid * H_TILE + i * EDGE_CH,
                                               EDGE_CH)])
            return carry

        lax.fori_loop(0, H_TILE // EDGE_CH, zcp, 0)
        plsc.subcore_barrier()

        def gathers(ci, b):
            @pl.when(cid == 0)
            def _():
                pltpu.async_copy(q0_hbm.at[sidx.at[ci]], qb[b], sq[b])
                pltpu.async_copy(r0_hbm.at[didx.at[ci]], rb[b], sr[b])

            @pl.when(cid == 1)
            def _():
                pltpu.async_copy(q1_hbm.at[sidx.at[ci]], qb[b], sq[b])
                pltpu.async_copy(r1_hbm.at[didx.at[ci]], rb[b], sr[b])

        for b in range(nb - lag):
            gathers(b, b)

        def body(g, carry):
            for b in range(nb):
                ci = g * nb + b
                pltpu.make_async_copy(q0_hbm.at[sidx.at[0]], qb[b],
                                      sq[b]).wait()
                pltpu.make_async_copy(r0_hbm.at[didx.at[0]], rb[b],
                                      sr[b]).wait()

                qbb, rbb = qb[b], rb[b]

                @plsc.parallel_loop(0, EDGE_CH, 1, unroll=8)
                def _(r):
                    for gg in range(COL // 16):
                        sl = pl.ds(gg * 16, 16)
                        rbb[r, sl] = jnp.maximum(qbb[r, sl] + rbb[r, sl], 0.0)

                pltpu.async_copy(rb[b], hacc.at[didx.at[ci]], ss[b], add=True)

                bf = (b + nb - lag) % nb
                nxt = ci + nb - lag

                @pl.when(jnp.logical_and(ci >= lag, nxt < EDGE_NCH))
                def _():
                    pltpu.make_async_copy(rb[bf], hacc.at[didx.at[0]],
                                          ss[bf]).wait()
                    gathers(nxt, bf)

                @pl.when(jnp.logical_and(ci < lag, nxt < EDGE_NCH))
                def _():
                    gathers(nxt, bf)
            return carry

        lax.fori_loop(0, n_g, body, 0)
        for b in range(nb):
            pltpu.make_async_copy(rb[b], hacc.at[didx.at[0]], ss[b]).wait()
        plsc.subcore_barrier()

        def ocp(i, carry):
            off = sid * H_TILE + i * EDGE_CH
            pltpu.sync_copy(hacc.at[pl.ds(off, EDGE_CH)],
                            out_hbm.at[pl.ds(off, EDGE_CH),
                                       pl.ds(cid * COL, COL)])
            return carry

        lax.fori_loop(0, H_TILE // EDGE_CH, ocp, 0)

    return k(q0, q1, r0, r1, src3, dst3)


def _lstm_body(h_ref, st_ref, wm2_ref, wih_ref, whh_ref, bg_ref,
               wo_ref, bo_ref, o_ref):
    sm = jnp.dot(h_ref[...], wm2_ref[...], preferred_element_type=jnp.float32)
    st = st_ref[...]
    gates = (jnp.dot(sm, wih_ref[...], preferred_element_type=jnp.float32)
             + jnp.dot(st[:, :HID], whh_ref[...],
                       preferred_element_type=jnp.float32)
             + bg_ref[...])
    i = jax.nn.sigmoid(gates[:, 0 * HID:1 * HID])
    f = jax.nn.sigmoid(gates[:, 1 * HID:2 * HID])
    g = jnp.tanh(gates[:, 2 * HID:3 * HID])
    o = jax.nn.sigmoid(gates[:, 3 * HID:4 * HID])
    c = f * st[:, HID:] + i * g
    sn = o * jnp.tanh(c)
    o_ref[...] = (jnp.dot(sn, wo_ref[...], preferred_element_type=jnp.float32)
                  + bo_ref[...])


def _lstm_head(hout, state_cat, Wm2, Wih, Whh, bg, Wo2, bo2):
    R = 1000
    return pl.pallas_call(
        _lstm_body,
        grid=(N_NODES // R,),
        in_specs=[
            pl.BlockSpec((R, HID), lambda i: (i, 0)),
            pl.BlockSpec((R, 2 * HID), lambda i: (i, 0)),
            pl.BlockSpec((HID, HID), lambda i: (0, 0)),
            pl.BlockSpec((HID, 4 * HID), lambda i: (0, 0)),
            pl.BlockSpec((HID, 4 * HID), lambda i: (0, 0)),
            pl.BlockSpec((1, 4 * HID), lambda i: (0, 0)),
            pl.BlockSpec((HID, 8), lambda i: (0, 0)),
            pl.BlockSpec((1, 8), lambda i: (0, 0)),
        ],
        out_specs=pl.BlockSpec((R, 8), lambda i: (i, 0)),
        out_shape=jax.ShapeDtypeStruct((N_NODES, 8), jnp.float32),
    )(hout, state_cat, Wm2, Wih, Whh, bg, Wo2, bo2)


def _pred_head(s8f, i0, i1):
    groups = PRED_PER_W // 16

    @functools.partial(
        pl.kernel,
        out_type=jax.ShapeDtypeStruct((PRED_PAD,), jnp.float32),
        mesh=plsc.VectorSubcoreMesh(**_MESH),
        compiler_params=pltpu.CompilerParams(needs_layout_passes=False),
        scratch_types=[
            pltpu.VMEM((N_NODES * 8,), jnp.float32),
            pltpu.VMEM((PRED_PER_W,), jnp.int32),
            pltpu.VMEM((PRED_PER_W,), jnp.int32),
            pltpu.VMEM((PRED_PER_W,), jnp.float32),
        ],
    )
    def k(s8_hbm, i0_hbm, i1_hbm, out_hbm, s8_v, i0_v, i1_v, ob):
        wid = lax.axis_index("s") * 2 + lax.axis_index("c")
        base = pl.multiple_of(wid * PRED_PER_W, 8)
        pltpu.sync_copy(s8_hbm, s8_v)
        pltpu.sync_copy(i0_hbm.at[pl.ds(base, PRED_PER_W)], i0_v)
        pltpu.sync_copy(i1_hbm.at[pl.ds(base, PRED_PER_W)], i1_v)

        def body(j, carry):
            sl = pl.ds(j * 16, 16)
            a = plsc.load_gather(s8_v, [i0_v[sl] * 8])
            b = plsc.load_gather(s8_v, [i1_v[sl] * 8 + 1])
            ob[sl] = a - b
            return carry

        lax.fori_loop(0, groups, body, 0)
        pltpu.sync_copy(ob, out_hbm.at[pl.ds(base, PRED_PER_W)])

    return k(s8f, i0, i1)


def kernel(A_pad, edges, node_idx_gnn, node_idx_feat, att_idx, node_dist,
           W_in, b_in, W_inc, b_inc, Wm1, bm1, Wm2, bm2,
           W_ih, W_hh, b_ih, b_hh, Wo, bo):
    f32 = jnp.float32
    nd = node_dist.reshape(-1).astype(jnp.int32)
    att = att_idx.astype(jnp.int32)

    Wc = jnp.concatenate([W_in, W_inc], axis=1)
    bc = jnp.concatenate([b_in, b_inc]).reshape(1, 2 * HID)
    A4 = A_pad.reshape(N_B, N_MAX, 1, N_MAX)
    table = _node_feats(A4, Wc, bc).reshape(T_ROWS, 2 * HID)

    nif = node_idx_feat.astype(jnp.int32)
    idx0 = jnp.where(nif == 0, ZROW, nif - 1)
    idx3 = jnp.concatenate(
        [idx0, jnp.full((NODE_PAD - N_NODES,), ZROW, jnp.int32)]
    ).reshape(NW, 8, 40)
    state_cat = _gather_rows(table, idx3)

    W0 = Wm1[:HID]
    Wqr = jnp.concatenate([
        jnp.concatenate([W0, Wm1[HID:HID + ATT],
                         Wm1[HID + 2 * ATT:HID + 2 * ATT + NDF]], axis=0),
        jnp.concatenate([-W0, Wm1[HID + ATT:HID + 2 * ATT],
                         Wm1[HID + 2 * ATT + NDF:]], axis=0),
    ], axis=1)
    bqr = jnp.concatenate([bm1, jnp.zeros((HID,), f32)]).reshape(1, 2 * HID)
    q0, q1, r0, r1 = _qr(state_cat, att.reshape(-1, 1), nd.reshape(-1, 1),
                         Wqr, bqr)

    pad_e = jnp.full((1, EDGE_PAD - N_EDGES), N_NODES, jnp.int32)
    edT = edges.T.astype(jnp.int32)
    src3 = jnp.concatenate([edT[0:1], pad_e], axis=1) \
              .reshape(16, EDGE_NCH, EDGE_CH)
    dst3 = jnp.concatenate([edT[1:2], pad_e], axis=1) \
              .reshape(16, EDGE_NCH, EDGE_CH)
    hout = _edge_accumulate(q0, q1, r0, r1, src3, dst3)

    bg = (b_ih + b_hh).reshape(1, 4 * HID)
    Wo2 = jnp.concatenate([Wo, Wo] + [jnp.zeros((HID, 1), f32)] * 6, axis=1)
    bo2 = jnp.concatenate([bo, jnp.zeros((7,), f32)]).reshape(1, 8)
    s8 = _lstm_head(hout, state_cat, Wm2, W_ih, W_hh, bg, Wo2, bo2)

    pad_p = jnp.zeros((1, PRED_PAD - N_PRED), jnp.int32)
    giT = node_idx_gnn.T.astype(jnp.int32)
    i0 = jnp.concatenate([giT[0:1], pad_p], axis=1).reshape(-1)
    i1 = jnp.concatenate([giT[1:2], pad_p], axis=1).reshape(-1)
    out = _pred_head(s8.reshape(-1), i0, i1)
    return out[:N_PRED].reshape(-1, 1)


# speedup vs baseline: 1.0767x; 1.0154x over previous
import functools

import jax
import jax.numpy as jnp
from jax import lax
from jax.experimental import pallas as pl
from jax.experimental.pallas import tpu as pltpu
from jax.experimental.pallas import tpu_sc as plsc

N_B = 16
N_MAX = 625
N_NODES = 10000
N_EDGES = 160000
N_PRED = 100000
HID = 128
ATT = 16
NDF = 16

NW = 32
NODE_PAD = 10240
T_ROWS = (N_B + 1) * N_MAX
ZROW = N_NODES + 304
COL = 64
QOFF = 11000
EDGE_CH = 128
EDGE_NCH = 80
EDGE_PAD = 16 * EDGE_NCH * EDGE_CH
H_ROWS = 10240
H_TILE = H_ROWS // 16
PRED_PER_W = 3200
PRED_PAD = NW * PRED_PER_W

_MESH = dict(core_axis_name="c", subcore_axis_name="s")


def _nf_body(a_ref, w_ref, b_ref, o_ref):
    i = pl.program_id(0)

    @pl.when(i < N_B)
    def _():
        o_ref[0] = (
            jnp.dot(a_ref[0, :, 0, :], w_ref[...],
                    preferred_element_type=jnp.float32)
            + b_ref[...]
        )

    @pl.when(i >= N_B)
    def _():
        o_ref[0] = jnp.zeros((N_MAX, 2 * HID), jnp.float32)


def _node_feats(A_pad, Wc, bc):
    return pl.pallas_call(
        _nf_body,
        grid=(N_B + 1,),
        in_specs=[
            pl.BlockSpec((1, N_MAX, 1, N_MAX),
                         lambda i: (jnp.minimum(i, N_B - 1), 0, 0, 0)),
            pl.BlockSpec((N_MAX, 2 * HID), lambda i: (0, 0)),
            pl.BlockSpec((1, 2 * HID), lambda i: (0, 0)),
        ],
        out_specs=pl.BlockSpec((1, N_MAX, 2 * HID), lambda i: (i, 0, 0)),
        out_shape=jax.ShapeDtypeStruct((N_B + 1, N_MAX, 2 * HID),
                                       jnp.float32),
    )(A_pad, Wc, bc)


def _gather_rows(table, idx3):
    per_w = NODE_PAD // NW
    ch = 40
    n_ch = per_w // ch
    nb = 4
    lag = 2

    @functools.partial(
        pl.kernel,
        out_type=jax.ShapeDtypeStruct((NODE_PAD, 2 * HID), jnp.float32),
        mesh=plsc.VectorSubcoreMesh(**_MESH),
        scratch_types=[
            pltpu.VMEM((n_ch, ch), jnp.int32),
            pltpu.VMEM((ch, 2 * HID), jnp.float32),
            pltpu.VMEM((ch, 2 * HID), jnp.float32),
            pltpu.VMEM((ch, 2 * HID), jnp.float32),
            pltpu.VMEM((ch, 2 * HID), jnp.float32),
            pltpu.SemaphoreType.DMA,
            pltpu.SemaphoreType.DMA,
            pltpu.SemaphoreType.DMA,
            pltpu.SemaphoreType.DMA,
        ],
    )
    def k(table_hbm, idx_hbm, out_hbm, idx_v, r0, r1, r2, r3,
          g0, g1, g2, g3):
        wid = lax.axis_index("s") * 2 + lax.axis_index("c")
        rows = [r0, r1, r2, r3]
        gsem = [g0, g1, g2, g3]
        pltpu.sync_copy(idx_hbm.at[wid], idx_v)
        for b in range(nb - lag):
            pltpu.async_copy(table_hbm.at[idx_v.at[b]], rows[b], gsem[b])

        def body(ci, carry):
            for b in range(nb):
                c = ci * nb + b
                pltpu.make_async_copy(table_hbm.at[idx_v.at[0]], rows[b],
                                      gsem[b]).wait()

                bf = (b + nb - lag) % nb
                nxt = c + nb - lag

                @pl.when(nxt < n_ch)
                def _():
                    pltpu.async_copy(table_hbm.at[idx_v.at[nxt]], rows[bf],
                                     gsem[bf])

                base = pl.multiple_of(wid * per_w + c * ch, 8)
                pltpu.sync_copy(rows[b], out_hbm.at[pl.ds(base, ch)])
            return carry

        lax.fori_loop(0, n_ch // nb, body, 0)

    return k(table, idx3)


def _qr_body(s_ref, ai_ref, ni_ref, w_ref, b_ref,
             q0_ref, q1_ref, r0_ref, r1_ref):
    i = pl.program_id(0)

    @pl.when(i < 10)
    def _():
        s = s_ref[...][:, :HID]
        rows = s.shape[0]
        oha = (lax.broadcasted_iota(jnp.int32, (rows, ATT), 1) == ai_ref[...])
        ohn = (lax.broadcasted_iota(jnp.int32, (rows, NDF), 1) == ni_ref[...])
        x = jnp.concatenate(
            [s, oha.astype(jnp.float32), ohn.astype(jnp.float32)], axis=1)
        qr = (jnp.dot(x, w_ref[...], preferred_element_type=jnp.float32)
              + b_ref[...])
        q0_ref[...] = qr[:, 0 * COL:1 * COL]
        q1_ref[...] = qr[:, 1 * COL:2 * COL]
        r0_ref[...] = qr[:, 2 * COL:3 * COL]
        r1_ref[...] = qr[:, 3 * COL:4 * COL]

    @pl.when(i >= 10)
    def _():
        q0_ref[...] = jnp.zeros_like(q0_ref)
        q1_ref[...] = jnp.zeros_like(q1_ref)
        r0_ref[...] = jnp.zeros_like(r0_ref)
        r1_ref[...] = jnp.zeros_like(r1_ref)


def _qr(state_cat, att2, nd2, Wqr, bqr):
    R = 1000
    K = HID + ATT + NDF
    qspec = pl.BlockSpec((R, COL), lambda i: (i, 0))
    qshape = jax.ShapeDtypeStruct((QOFF, COL), jnp.float32)
    return pl.pallas_call(
        _qr_body,
        grid=(QOFF // R,),
        in_specs=[
            pl.BlockSpec((R, 2 * HID), lambda i: (jnp.minimum(i, 9), 0)),
            pl.BlockSpec((R, 1), lambda i: (jnp.minimum(i, 9), 0)),
            pl.BlockSpec((R, 1), lambda i: (jnp.minimum(i, 9), 0)),
            pl.BlockSpec((K, 2 * HID), lambda i: (0, 0)),
            pl.BlockSpec((1, 2 * HID), lambda i: (0, 0)),
        ],
        out_specs=[qspec, qspec, qspec, qspec],
        out_shape=[qshape, qshape, qshape, qshape],
    )(state_cat, att2, nd2, Wqr, bqr)


def _edge_accumulate(q0, q1, r0, r1, src3, dst3):
    nb = 4
    lag = 2
    n_g = EDGE_NCH // nb

    @functools.partial(
        pl.kernel,
        out_type=jax.ShapeDtypeStruct((H_ROWS, HID), jnp.float32),
        mesh=plsc.VectorSubcoreMesh(**_MESH),
        compiler_params=pltpu.CompilerParams(use_tc_tiling_on_sc=False),
        scratch_types=[
            pltpu.VMEM((EDGE_NCH, EDGE_CH), jnp.int32),
            pltpu.VMEM((EDGE_NCH, EDGE_CH), jnp.int32),
            pltpu.VMEM((EDGE_CH, COL), jnp.float32),
            pltpu.VMEM((EDGE_CH, COL), jnp.float32),
            pltpu.VMEM((EDGE_CH, COL), jnp.float32),
            pltpu.VMEM((EDGE_CH, COL), jnp.float32),
            pltpu.VMEM((EDGE_CH, COL), jnp.float32),
            pltpu.VMEM((EDGE_CH, COL), jnp.float32),
            pltpu.VMEM((EDGE_CH, COL), jnp.float32),
            pltpu.VMEM((EDGE_CH, COL), jnp.float32),
            pltpu.VMEM_SHARED((H_ROWS, COL), jnp.float32),
            pltpu.SemaphoreType.DMA,
            pltpu.SemaphoreType.DMA,
            pltpu.SemaphoreType.DMA,
            pltpu.SemaphoreType.DMA,
            pltpu.SemaphoreType.DMA,
            pltpu.SemaphoreType.DMA,
            pltpu.SemaphoreType.DMA,
            pltpu.SemaphoreType.DMA,
        ],
    )
    def k(q0_hbm, q1_hbm, r0_hbm, r1_hbm, src_hbm, dst_hbm, out_hbm,
          sidx, didx,
          qb0, qb1, qb2, qb3, rb0, rb1, rb2, rb3, hacc,
          sq0, sq1, sq2, sq3, sr0, sr1, sr2, sr3):
        qb = [qb0, qb1, qb2, qb3]
        rb = [rb0, rb1, rb2, rb3]
        sq = [sq0, sq1, sq2, sq3]
        sr = [sr0, sr1, sr2, sr3]
        cid = lax.axis_index("c")
        sid = lax.axis_index("s")

        pltpu.sync_copy(src_hbm.at[sid], sidx)
        pltpu.sync_copy(dst_hbm.at[sid], didx)

        zero = jnp.zeros((16,), jnp.float32)

        def zrow(r, carry):
            for g in range(COL // 16):
                qb0[r, pl.ds(g * 16, 16)] = zero
            return carry

        lax.fori_loop(0, EDGE_CH, zrow, 0)

        def zcp(i, carry):
            pltpu.sync_copy(qb0, hacc.at[pl.ds(sid * H_TILE + i * EDGE_CH,
                                               EDGE_CH)])
            return carry

        lax.fori_loop(0, H_TILE // EDGE_CH, zcp, 0)
        plsc.subcore_barrier()

        def gathers(ci, b):
            @pl.when(cid == 0)
            def _():
                pltpu.async_copy(q0_hbm.at[sidx.at[ci]], qb[b], sq[b])
                pltpu.async_copy(r0_hbm.at[didx.at[ci]], rb[b], sr[b])

            @pl.when(cid == 1)
            def _():
                pltpu.async_copy(q1_hbm.at[sidx.at[ci]], qb[b], sq[b])
                pltpu.async_copy(r1_hbm.at[didx.at[ci]], rb[b], sr[b])

        for b in range(nb - lag):
            gathers(b, b)

        def body(g, carry):
            for b in range(nb):
                ci = g * nb + b
                pltpu.make_async_copy(q0_hbm.at[sidx.at[0]], qb[b],
                                      sq[b]).wait()
                pltpu.make_async_copy(r0_hbm.at[didx.at[0]], rb[b],
                                      sr[b]).wait()

                qbb, rbb = qb[b], rb[b]

                @plsc.parallel_loop(0, EDGE_CH, 1, unroll=8)
                def _(r):
                    for gg in range(COL // 16):
                        sl = pl.ds(gg * 16, 16)
                        rbb[r, sl] = jnp.maximum(qbb[r, sl] + rbb[r, sl], 0.0)

                bf = (b + nb - lag) % nb
                nxt = ci + nb - lag

                @pl.when(nxt < EDGE_NCH)
                def _():
                    gathers(nxt, bf)

                pltpu.sync_copy(rb[b], hacc.at[didx.at[ci]], add=True)
            return carry

        lax.fori_loop(0, n_g, body, 0)
        plsc.subcore_barrier()

        def ocp(i, carry):
            off = sid * H_TILE + i * EDGE_CH
            pltpu.sync_copy(hacc.at[pl.ds(off, EDGE_CH)],
                            out_hbm.at[pl.ds(off, EDGE_CH),
                                       pl.ds(cid * COL, COL)])
            return carry

        lax.fori_loop(0, H_TILE // EDGE_CH, ocp, 0)

    return k(q0, q1, r0, r1, src3, dst3)


def _lstm_body(h_ref, st_ref, wm2_ref, wih_ref, whh_ref, bg_ref,
               wo_ref, bo_ref, o_ref):
    sm = jnp.dot(h_ref[...], wm2_ref[...], preferred_element_type=jnp.float32)
    st = st_ref[...]
    gates = (jnp.dot(sm, wih_ref[...], preferred_element_type=jnp.float32)
             + jnp.dot(st[:, :HID], whh_ref[...],
                       preferred_element_type=jnp.float32)
             + bg_ref[...])
    i = jax.nn.sigmoid(gates[:, 0 * HID:1 * HID])
    f = jax.nn.sigmoid(gates[:, 1 * HID:2 * HID])
    g = jnp.tanh(gates[:, 2 * HID:3 * HID])
    o = jax.nn.sigmoid(gates[:, 3 * HID:4 * HID])
    c = f * st[:, HID:] + i * g
    sn = o * jnp.tanh(c)
    o_ref[...] = (jnp.dot(sn, wo_ref[...], preferred_element_type=jnp.float32)
                  + bo_ref[...])


def _lstm_head(hout, state_cat, Wm2, Wih, Whh, bg, Wo2, bo2):
    R = 1000
    return pl.pallas_call(
        _lstm_body,
        grid=(N_NODES // R,),
        in_specs=[
            pl.BlockSpec((R, HID), lambda i: (i, 0)),
            pl.BlockSpec((R, 2 * HID), lambda i: (i, 0)),
            pl.BlockSpec((HID, HID), lambda i: (0, 0)),
            pl.BlockSpec((HID, 4 * HID), lambda i: (0, 0)),
            pl.BlockSpec((HID, 4 * HID), lambda i: (0, 0)),
            pl.BlockSpec((1, 4 * HID), lambda i: (0, 0)),
            pl.BlockSpec((HID, 8), lambda i: (0, 0)),
            pl.BlockSpec((1, 8), lambda i: (0, 0)),
        ],
        out_specs=pl.BlockSpec((R, 8), lambda i: (i, 0)),
        out_shape=jax.ShapeDtypeStruct((N_NODES, 8), jnp.float32),
    )(hout, state_cat, Wm2, Wih, Whh, bg, Wo2, bo2)


def _pred_head(s8f, i0, i1):
    groups = PRED_PER_W // 16

    @functools.partial(
        pl.kernel,
        out_type=jax.ShapeDtypeStruct((PRED_PAD,), jnp.float32),
        mesh=plsc.VectorSubcoreMesh(**_MESH),
        compiler_params=pltpu.CompilerParams(needs_layout_passes=False),
        scratch_types=[
            pltpu.VMEM((N_NODES * 8,), jnp.float32),
            pltpu.VMEM((PRED_PER_W,), jnp.int32),
            pltpu.VMEM((PRED_PER_W,), jnp.int32),
            pltpu.VMEM((PRED_PER_W,), jnp.float32),
        ],
    )
    def k(s8_hbm, i0_hbm, i1_hbm, out_hbm, s8_v, i0_v, i1_v, ob):
        wid = lax.axis_index("s") * 2 + lax.axis_index("c")
        base = pl.multiple_of(wid * PRED_PER_W, 8)
        pltpu.sync_copy(s8_hbm, s8_v)
        pltpu.sync_copy(i0_hbm.at[pl.ds(base, PRED_PER_W)], i0_v)
        pltpu.sync_copy(i1_hbm.at[pl.ds(base, PRED_PER_W)], i1_v)

        def body(j, carry):
            sl = pl.ds(j * 16, 16)
            a = plsc.load_gather(s8_v, [i0_v[sl] * 8])
            b = plsc.load_gather(s8_v, [i1_v[sl] * 8 + 1])
            ob[sl] = a - b
            return carry

        lax.fori_loop(0, groups, body, 0)
        pltpu.sync_copy(ob, out_hbm.at[pl.ds(base, PRED_PER_W)])

    return k(s8f, i0, i1)


def kernel(A_pad, edges, node_idx_gnn, node_idx_feat, att_idx, node_dist,
           W_in, b_in, W_inc, b_inc, Wm1, bm1, Wm2, bm2,
           W_ih, W_hh, b_ih, b_hh, Wo, bo):
    f32 = jnp.float32
    nd = node_dist.reshape(-1).astype(jnp.int32)
    att = att_idx.astype(jnp.int32)

    Wc = jnp.concatenate([W_in, W_inc], axis=1)
    bc = jnp.concatenate([b_in, b_inc]).reshape(1, 2 * HID)
    A4 = A_pad.reshape(N_B, N_MAX, 1, N_MAX)
    table = _node_feats(A4, Wc, bc).reshape(T_ROWS, 2 * HID)

    nif = node_idx_feat.astype(jnp.int32)
    idx0 = jnp.where(nif == 0, ZROW, nif - 1)
    idx3 = jnp.concatenate(
        [idx0, jnp.full((NODE_PAD - N_NODES,), ZROW, jnp.int32)]
    ).reshape(NW, 8, 40)
    state_cat = _gather_rows(table, idx3)

    W0 = Wm1[:HID]
    Wqr = jnp.concatenate([
        jnp.concatenate([W0, Wm1[HID:HID + ATT],
                         Wm1[HID + 2 * ATT:HID + 2 * ATT + NDF]], axis=0),
        jnp.concatenate([-W0, Wm1[HID + ATT:HID + 2 * ATT],
                         Wm1[HID + 2 * ATT + NDF:]], axis=0),
    ], axis=1)
    bqr = jnp.concatenate([bm1, jnp.zeros((HID,), f32)]).reshape(1, 2 * HID)
    q0, q1, r0, r1 = _qr(state_cat, att.reshape(-1, 1), nd.reshape(-1, 1),
                         Wqr, bqr)

    pad_e = jnp.full((1, EDGE_PAD - N_EDGES), N_NODES, jnp.int32)
    edT = edges.T.astype(jnp.int32)
    src3 = jnp.concatenate([edT[0:1], pad_e], axis=1) \
              .reshape(16, EDGE_NCH, EDGE_CH)
    dst3 = jnp.concatenate([edT[1:2], pad_e], axis=1) \
              .reshape(16, EDGE_NCH, EDGE_CH)
    hout = _edge_accumulate(q0, q1, r0, r1, src3, dst3)

    bg = (b_ih + b_hh).reshape(1, 4 * HID)
    Wo2 = jnp.concatenate([Wo, Wo] + [jnp.zeros((HID, 1), f32)] * 6, axis=1)
    bo2 = jnp.concatenate([bo, jnp.zeros((7,), f32)]).reshape(1, 8)
    s8 = _lstm_head(hout, state_cat, Wm2, W_ih, W_hh, bg, Wo2, bo2)

    pad_p = jnp.zeros((1, PRED_PAD - N_PRED), jnp.int32)
    giT = node_idx_gnn.T.astype(jnp.int32)
    i0 = jnp.concatenate([giT[0:1], pad_p], axis=1).reshape(-1)
    i1 = jnp.concatenate([giT[1:2], pad_p], axis=1).reshape(-1)
    out = _pred_head(s8.reshape(-1), i0, i1)
    return out[:N_PRED].reshape(-1, 1)
